# baseline (speedup 1.0000x reference)
import functools

import jax
import jax.numpy as jnp
from jax import lax
from jax.experimental import pallas as pl
from jax.experimental.pallas import tpu as pltpu
from jax.experimental.pallas import tpu_sc as plsc

B, L, T = 2, 1, 4096
V, D = 4096, 4096
NT = 10
R = NT + T

NC, NS, LANES = 2, 16, 16
NW = NC * NS
ROWS_PER_W = (B * T) // NW
CH = 16
NCHUNK = ROWS_PER_W // CH

_mesh = plsc.VectorSubcoreMesh(core_axis_name="c", subcore_axis_name="s")


@functools.partial(
    pl.kernel,
    out_type=jax.ShapeDtypeStruct((B * R, D), jnp.float32),
    mesh=_mesh,
    scratch_types=[
        pltpu.VMEM((ROWS_PER_W,), jnp.int32),
        pltpu.VMEM((CH, D), jnp.float32),
        pltpu.VMEM((LANES,), jnp.float32),
        pltpu.VMEM((D,), jnp.float32),
        pltpu.VMEM((D,), jnp.float32),
        pltpu.SemaphoreType.DMA,
        pltpu.SemaphoreType.DMA,
    ],
)
def _mpt_sc(idx_hbm, table_hbm, sp_hbm, u16_hbm, v_hbm, out_hbm,
            idx_v, gbuf, u_v, v_v, row_v, gsem, osem):
    cid = lax.axis_index("c")
    sid = lax.axis_index("s")
    wid = sid * NC + cid
    io = lax.iota(jnp.int32, LANES)

    b = wid // (NW // B)
    ob = b * R + NT + (wid - b * (NW // B)) * ROWS_PER_W

    pltpu.sync_copy(idx_hbm.at[pl.ds(wid * ROWS_PER_W, ROWS_PER_W)], idx_v)

    @pl.when(wid < B * NT)
    def _prompt():
        bp = wid // NT
        n = wid - bp * NT
        pltpu.sync_copy(u16_hbm.at[pl.ds(n * LANES, LANES)], u_v)
        pltpu.sync_copy(v_hbm, v_v)
        pltpu.sync_copy(sp_hbm.at[pl.ds(n * D, D)], row_v)
        un = u_v[...]

        def pbody(j, carry):
            s = pl.ds(j * LANES, LANES)
            gbuf[0, s] = un * v_v[s] * row_v[s]
            return carry

        lax.fori_loop(0, D // LANES, pbody, 0)
        dst = jnp.where(io == 0, bp * R + n, ob + io)
        pltpu.async_copy(gbuf, out_hbm.at[dst], osem).wait()

    for i in range(NCHUNK):
        tok = idx_v[pl.ds(i * CH, CH)]
        pltpu.async_copy(table_hbm.at[tok], gbuf, gsem).wait()
        pltpu.async_copy(gbuf, out_hbm.at[ob + i * CH + io], osem).wait()


def kernel(tokens, wte_weight, shared_prompt, u, v):
    idx = tokens.reshape(B * T).astype(jnp.int32)
    u16 = jnp.tile(u.reshape(NT, 1), (1, LANES)).reshape(NT * LANES)
    out = _mpt_sc(idx, wte_weight, shared_prompt.reshape(NT * D), u16,
                  v.reshape(D))
    return out.reshape(B, R, D)[:, None]


# speedup vs baseline: 1.0208x; 1.0208x over previous
import functools

import jax
import jax.numpy as jnp
from jax import lax
from jax.experimental import pallas as pl
from jax.experimental.pallas import tpu as pltpu
from jax.experimental.pallas import tpu_sc as plsc

B, L, T = 2, 1, 4096
V, D = 4096, 4096
NT = 10
R = NT + T

NC, NS, LANES = 2, 16, 16
NW = NC * NS
ROWS_PER_W = (B * T) // NW
CH = 16
DH = D // 2
NCHUNK = ROWS_PER_W // CH
NXFER = 2 * NCHUNK

_mesh = plsc.VectorSubcoreMesh(core_axis_name="c", subcore_axis_name="s")


@functools.partial(
    pl.kernel,
    out_type=jax.ShapeDtypeStruct((B * R, D), jnp.float32),
    mesh=_mesh,
    scratch_types=[
        pltpu.VMEM((ROWS_PER_W,), jnp.int32),
        pltpu.VMEM((2, CH, DH), jnp.float32),
        pltpu.VMEM((LANES,), jnp.float32),
        pltpu.VMEM((D,), jnp.float32),
        pltpu.VMEM((D,), jnp.float32),
        pltpu.SemaphoreType.DMA,
        pltpu.SemaphoreType.DMA,
        pltpu.SemaphoreType.DMA,
        pltpu.SemaphoreType.DMA,
    ],
)
def _mpt_sc(idx_hbm, table_hbm, sp_hbm, u16_hbm, v_hbm, out_hbm,
            idx_v, gbuf, u_v, v_v, row_v, gsem0, gsem1, osem0, osem1):
    cid = lax.axis_index("c")
    sid = lax.axis_index("s")
    wid = sid * NC + cid
    io = lax.iota(jnp.int32, LANES)
    gsem = (gsem0, gsem1)
    osem = (osem0, osem1)

    b = wid // (NW // B)
    ob = b * R + NT + (wid - b * (NW // B)) * ROWS_PER_W

    pltpu.sync_copy(idx_hbm.at[pl.ds(wid * ROWS_PER_W, ROWS_PER_W)], idx_v)

    @pl.when(wid < B * NT)
    def _prompt():
        bp = wid // NT
        n = wid - bp * NT
        pltpu.sync_copy(u16_hbm.at[pl.ds(n * LANES, LANES)], u_v)
        pltpu.sync_copy(v_hbm, v_v)
        pltpu.sync_copy(sp_hbm.at[pl.ds(n * D, D)], row_v)
        un = u_v[...]

        def pbody(j, carry):
            s0 = pl.ds(j * LANES, LANES)
            s1 = pl.ds(DH + j * LANES, LANES)
            gbuf[0, 0, s0] = un * v_v[s0] * row_v[s0]
            gbuf[1, 0, s0] = un * v_v[s1] * row_v[s1]
            return carry

        lax.fori_loop(0, DH // LANES, pbody, 0)
        dst = jnp.where(io == 0, bp * R + n, ob + io)
        for h in range(2):
            pltpu.async_copy(gbuf.at[h],
                             out_hbm.at[dst, pl.ds(h * DH, DH)],
                             osem[h]).wait()

    def _gather(t_chunk, bb):
        tok = idx_v[pl.ds(t_chunk * CH, CH)]
        src = table_hbm.at[tok, pl.ds((1 - bb) * DH, DH)]
        pltpu.async_copy(src, gbuf.at[1 - bb], gsem[1 - bb])

    def _scatter(t_chunk, bb):
        dst = out_hbm.at[ob + t_chunk * CH + io, pl.ds(bb * DH, DH)]
        pltpu.async_copy(gbuf.at[bb], dst, osem[bb])

    pltpu.async_copy(table_hbm.at[idx_v[pl.ds(0, CH)], pl.ds(0, DH)],
                     gbuf.at[0], gsem[0])

    def body(i, carry):
        t0 = 2 * i
        for bb in range(2):
            t = t0 + bb
            pltpu.make_async_copy(table_hbm.at[io, pl.ds(0, DH)],
                                  gbuf.at[bb], gsem[bb]).wait()
            _scatter(i, bb)

            @pl.when(t >= 1)
            def _():
                pltpu.make_async_copy(
                    gbuf.at[1 - bb],
                    out_hbm.at[io, pl.ds(0, DH)], osem[1 - bb]).wait()

            @pl.when(t + 1 < NXFER)
            def _():
                _gather(i + bb, bb)
        return carry

    lax.fori_loop(0, NCHUNK, body, 0)
    pltpu.make_async_copy(gbuf.at[1], out_hbm.at[io, pl.ds(0, DH)],
                          osem[1]).wait()


def kernel(tokens, wte_weight, shared_prompt, u, v):
    idx = tokens.reshape(B * T).astype(jnp.int32)
    u16 = jnp.tile(u.reshape(NT, 1), (1, LANES)).reshape(NT * LANES)
    out = _mpt_sc(idx, wte_weight, shared_prompt.reshape(NT * D), u16,
                  v.reshape(D))
    return out.reshape(B, R, D)[:, None]
